# Initial kernel scaffold; baseline (speedup 1.0000x reference)
import jax
import jax.numpy as jnp
from jax.experimental import pallas as pl


def kernel(edge_index, edge_feats, sink_node_mask, W, b):
    raise NotImplementedError("write your pallas kernel here")


# speedup vs baseline: 75.3513x; 75.3513x over previous
import functools

import jax
import jax.numpy as jnp
from jax import lax
from jax.experimental import pallas as pl
from jax.experimental.pallas import tpu as pltpu
from jax.experimental.pallas import tpu_sc as plsc

_N = 100000
_E = 3200000
_D = 16
_ITERS = 60
_NS = 16
_NP = 100352
_NT = _NP // _NS
_ET = _E // _NS
_C = 2000
_NCHUNK = _ET // _C
_VPC = _C // 16
_EB = 25600


def _encoder_body(feats8_ref, m_ref, b_ref, rew_ref, exw_ref):
    x = jnp.dot(feats8_ref[...], m_ref[...],
                preferred_element_type=jnp.float32,
                precision=lax.Precision.HIGHEST)
    x = x + b_ref[0, 0]
    r = -jax.nn.softplus(x)
    rew_ref[...] = r
    exw_ref[...] = jnp.exp(r)


def _log_body(z_ref, out_ref):
    out_ref[...] = jnp.log(z_ref[...])


def _fp_body(src_hbm, dst_hbm, w_hbm, sinkf_hbm,
             z_hbm, probs_hbm,
             srcv, dstv, wv, zdv, zsv, tv, sfv, av, sem,
             zbuf, zacc):
    tid = lax.axis_index("s")
    nbase = tid * _NT
    ebase = tid * _ET

    pltpu.sync_copy(sinkf_hbm.at[pl.ds(nbase, _NT)], sfv)
    pltpu.sync_copy(sfv, zbuf.at[pl.ds(nbase, _NT)])

    def _zero_av(i, c):
        av[pl.ds(i * 16, 16)] = jnp.zeros((16,), jnp.float32)
        return c

    lax.fori_loop(0, _NT // 16, _zero_av, 0)
    pltpu.sync_copy(av, zacc.at[pl.ds(nbase, _NT)])
    plsc.subcore_barrier()

    def _iter(it, c):
        def _chunk(k, cc):
            off = ebase + k * _C
            pltpu.sync_copy(dst_hbm.at[pl.ds(off, _C)], dstv)
            pltpu.sync_copy(w_hbm.at[pl.ds(off, _C)], wv)
            pltpu.async_copy(zbuf.at[dstv], zdv, sem).wait()

            def _mul(i, c2):
                s = pl.ds(i * 16, 16)
                tv[s] = wv[s] * zdv[s]
                return c2

            lax.fori_loop(0, _VPC, _mul, 0)
            pltpu.sync_copy(src_hbm.at[pl.ds(off, _C)], srcv)
            pltpu.async_copy(tv, zacc.at[srcv], sem, add=True).wait()
            return cc

        lax.fori_loop(0, _NCHUNK, _chunk, 0)
        plsc.subcore_barrier()

        pltpu.sync_copy(zacc.at[pl.ds(nbase, _NT)], av)

        def _fin(i, c2):
            s = pl.ds(i * 16, 16)
            sf = sfv[s]
            av[s] = sf + (1.0 - sf) * av[s]
            return c2

        lax.fori_loop(0, _NT // 16, _fin, 0)
        pltpu.sync_copy(av, zbuf.at[pl.ds(nbase, _NT)])
        lax.fori_loop(0, _NT // 16, _zero_av, 0)
        pltpu.sync_copy(av, zacc.at[pl.ds(nbase, _NT)])
        plsc.subcore_barrier()
        return c

    lax.fori_loop(0, _ITERS, _iter, 0)

    pltpu.sync_copy(zbuf.at[pl.ds(nbase, _NT)], z_hbm.at[pl.ds(nbase, _NT)])

    def _pchunk(k, c):
        off = ebase + k * _C
        pltpu.sync_copy(dst_hbm.at[pl.ds(off, _C)], dstv)
        pltpu.sync_copy(src_hbm.at[pl.ds(off, _C)], srcv)
        pltpu.sync_copy(w_hbm.at[pl.ds(off, _C)], wv)
        pltpu.async_copy(zbuf.at[dstv], zdv, sem).wait()
        pltpu.async_copy(zbuf.at[srcv], zsv, sem).wait()

        def _p(i, c2):
            s = pl.ds(i * 16, 16)
            tv[s] = wv[s] * zdv[s] / zsv[s]
            return c2

        lax.fori_loop(0, _VPC, _p, 0)
        pltpu.sync_copy(tv, probs_hbm.at[pl.ds(off, _C)])
        return c

    lax.fori_loop(0, _NCHUNK, _pchunk, 0)


def kernel(edge_index, edge_feats, sink_node_mask, W, b):
    src = edge_index[0]
    dst = edge_index[1]
    sinkf = jnp.pad(sink_node_mask.astype(jnp.float32), (0, _NP - _N))

    feats8 = edge_feats.reshape(_E // 8, 128)
    m = jnp.kron(jnp.eye(8, dtype=jnp.float32), W)
    eb8 = 3200
    rew8, exw8 = pl.pallas_call(
        _encoder_body,
        grid=(_E // 8 // eb8,),
        in_specs=[
            pl.BlockSpec((eb8, 128), lambda i: (i, 0)),
            pl.BlockSpec((128, 8), lambda i: (0, 0)),
            pl.BlockSpec((1, 1), lambda i: (0, 0)),
        ],
        out_specs=[
            pl.BlockSpec((eb8, 8), lambda i: (i, 0)),
            pl.BlockSpec((eb8, 8), lambda i: (i, 0)),
        ],
        out_shape=[
            jax.ShapeDtypeStruct((_E // 8, 8), jnp.float32),
            jax.ShapeDtypeStruct((_E // 8, 8), jnp.float32),
        ],
    )(feats8, m, b.reshape(1, 1))
    rewards = rew8.reshape(_E)
    exp_rewards = exw8.reshape(_E)

    mesh = plsc.VectorSubcoreMesh(
        core_axis_name="c", subcore_axis_name="s", num_cores=1)
    fp = pl.kernel(
        _fp_body,
        out_type=[
            jax.ShapeDtypeStruct((_NP,), jnp.float32),
            jax.ShapeDtypeStruct((_E,), jnp.float32),
        ],
        mesh=mesh,
        scratch_types=[
            pltpu.VMEM((_C,), jnp.int32),
            pltpu.VMEM((_C,), jnp.int32),
            pltpu.VMEM((_C,), jnp.float32),
            pltpu.VMEM((_C,), jnp.float32),
            pltpu.VMEM((_C,), jnp.float32),
            pltpu.VMEM((_C,), jnp.float32),
            pltpu.VMEM((_NT,), jnp.float32),
            pltpu.VMEM((_NT,), jnp.float32),
            pltpu.SemaphoreType.DMA,
            pltpu.VMEM_SHARED((_NP,), jnp.float32),
            pltpu.VMEM_SHARED((_NP,), jnp.float32),
        ],
    )
    z_pad, edge_probs = fp(src, dst, exp_rewards, sinkf)

    values = pl.pallas_call(
        _log_body,
        out_shape=jax.ShapeDtypeStruct((784, 128), jnp.float32),
    )(z_pad.reshape(784, 128)).reshape(_NP)[:_N]

    return rewards, values, edge_probs


# speedup vs baseline: 170.5463x; 2.2634x over previous
import functools

import jax
import jax.numpy as jnp
from jax import lax
from jax.experimental import pallas as pl
from jax.experimental.pallas import tpu as pltpu
from jax.experimental.pallas import tpu_sc as plsc

_N = 100000
_E = 3200000
_D = 16
_ITERS = 60
_NS = 16
_NP = 100352
_NT = _NP // _NS
_ET = _E // _NS
_C = 10000
_NCHUNK = _ET // _C
_VPC = _C // 16
_EB = 25600


def _encoder_body(feats8_ref, m_ref, b_ref, rew_ref, exw_ref):
    x = jnp.dot(feats8_ref[...], m_ref[...],
                preferred_element_type=jnp.float32,
                precision=lax.Precision.HIGHEST)
    x = x + b_ref[0, 0]
    r = -jax.nn.softplus(x)
    rew_ref[...] = r
    exw_ref[...] = jnp.exp(r)


def _log_body(z_ref, out_ref):
    out_ref[...] = jnp.log(z_ref[...])


def _fp_body(src_hbm, dst_hbm, w_hbm, sinkf_hbm,
             z_hbm, probs_hbm,
             srcA, dstA, wA, zdA, tvA,
             srcB, dstB, wB, zdB, tvB,
             sfv, av,
             semg, semlA, semlB, semsA, semsB,
             zbuf, zacc):
    tid = lax.axis_index("s")
    nbase = tid * _NT
    ebase = tid * _ET

    def _lin_issue(off, sv, dv, wvv, sem):
        pltpu.async_copy(src_hbm.at[pl.ds(off, _C)], sv, sem)
        pltpu.async_copy(dst_hbm.at[pl.ds(off, _C)], dv, sem)
        pltpu.async_copy(w_hbm.at[pl.ds(off, _C)], wvv, sem)

    def _lin_wait(off, sv, dv, wvv, sem):
        pltpu.make_async_copy(src_hbm.at[pl.ds(off, _C)], sv, sem).wait()
        pltpu.make_async_copy(dst_hbm.at[pl.ds(off, _C)], dv, sem).wait()
        pltpu.make_async_copy(w_hbm.at[pl.ds(off, _C)], wvv, sem).wait()

    def _mul_into(tvv, wvv, zdv_):
        def _mul(i, c2):
            s = pl.ds(i * 16, 16)
            tvv[s] = wvv[s] * zdv_[s]
            return c2

        lax.fori_loop(0, _VPC, _mul, 0)

    pltpu.sync_copy(sinkf_hbm.at[pl.ds(nbase, _NT)], sfv)
    pltpu.sync_copy(sfv, zbuf.at[pl.ds(nbase, _NT)])

    def _zero_av(i, c):
        av[pl.ds(i * 16, 16)] = jnp.zeros((16,), jnp.float32)
        return c

    lax.fori_loop(0, _NT // 16, _zero_av, 0)
    pltpu.sync_copy(av, zacc.at[pl.ds(nbase, _NT)])
    plsc.subcore_barrier()

    def _iter(it, c):
        _lin_issue(ebase, srcA, dstA, wA, semlA)

        def _chunk2(j, cc):
            kA = 2 * j
            kB = 2 * j + 1
            offA = ebase + kA * _C
            offB = ebase + kB * _C
            _lin_wait(offA, srcA, dstA, wA, semlA)
            gA = pltpu.async_copy(zbuf.at[dstA], zdA, semg)

            @pl.when(j > 0)
            def _():
                pltpu.make_async_copy(tvB, zacc.at[srcB], semsB).wait()

            _lin_issue(offB, srcB, dstB, wB, semlB)
            gA.wait()
            _mul_into(tvA, wA, zdA)
            dA = pltpu.async_copy(tvA, zacc.at[srcA], semsA, add=True)
            _lin_wait(offB, srcB, dstB, wB, semlB)
            gB = pltpu.async_copy(zbuf.at[dstB], zdB, semg)
            dA.wait()

            @pl.when(kB + 1 < _NCHUNK)
            def _():
                _lin_issue(ebase + (kB + 1) * _C, srcA, dstA, wA, semlA)

            gB.wait()
            _mul_into(tvB, wB, zdB)
            pltpu.async_copy(tvB, zacc.at[srcB], semsB, add=True)
            return cc

        lax.fori_loop(0, _NCHUNK // 2, _chunk2, 0)
        pltpu.make_async_copy(tvB, zacc.at[srcB], semsB).wait()
        plsc.subcore_barrier()

        pltpu.sync_copy(zacc.at[pl.ds(nbase, _NT)], av)

        def _fin(i, c2):
            s = pl.ds(i * 16, 16)
            sf = sfv[s]
            av[s] = sf + (1.0 - sf) * av[s]
            return c2

        lax.fori_loop(0, _NT // 16, _fin, 0)
        pltpu.sync_copy(av, zbuf.at[pl.ds(nbase, _NT)])
        lax.fori_loop(0, _NT // 16, _zero_av, 0)
        pltpu.sync_copy(av, zacc.at[pl.ds(nbase, _NT)])
        plsc.subcore_barrier()
        return c

    lax.fori_loop(0, _ITERS, _iter, 0)

    pltpu.sync_copy(zbuf.at[pl.ds(nbase, _NT)], z_hbm.at[pl.ds(nbase, _NT)])

    def _pchunk(k, c):
        off = ebase + k * _C
        pltpu.async_copy(dst_hbm.at[pl.ds(off, _C)], dstA, semlA)
        pltpu.async_copy(src_hbm.at[pl.ds(off, _C)], srcA, semlA)
        pltpu.async_copy(w_hbm.at[pl.ds(off, _C)], wA, semlA)
        pltpu.make_async_copy(dst_hbm.at[pl.ds(off, _C)], dstA, semlA).wait()
        pltpu.make_async_copy(src_hbm.at[pl.ds(off, _C)], srcA, semlA).wait()
        pltpu.make_async_copy(w_hbm.at[pl.ds(off, _C)], wA, semlA).wait()
        pltpu.async_copy(zbuf.at[dstA], zdA, semg).wait()
        pltpu.async_copy(zbuf.at[srcA], zdB, semg).wait()

        def _p(i, c2):
            s = pl.ds(i * 16, 16)
            tvA[s] = wA[s] * zdA[s] / zdB[s]
            return c2

        lax.fori_loop(0, _VPC, _p, 0)
        pltpu.sync_copy(tvA, probs_hbm.at[pl.ds(off, _C)])
        return c

    lax.fori_loop(0, _NCHUNK, _pchunk, 0)


def kernel(edge_index, edge_feats, sink_node_mask, W, b):
    src = edge_index[0]
    dst = edge_index[1]
    sinkf = jnp.pad(sink_node_mask.astype(jnp.float32), (0, _NP - _N))

    feats8 = edge_feats.reshape(_E // 8, 128)
    m = jnp.kron(jnp.eye(8, dtype=jnp.float32), W)
    eb8 = 3200
    rew8, exw8 = pl.pallas_call(
        _encoder_body,
        grid=(_E // 8 // eb8,),
        in_specs=[
            pl.BlockSpec((eb8, 128), lambda i: (i, 0)),
            pl.BlockSpec((128, 8), lambda i: (0, 0)),
            pl.BlockSpec((1, 1), lambda i: (0, 0)),
        ],
        out_specs=[
            pl.BlockSpec((eb8, 8), lambda i: (i, 0)),
            pl.BlockSpec((eb8, 8), lambda i: (i, 0)),
        ],
        out_shape=[
            jax.ShapeDtypeStruct((_E // 8, 8), jnp.float32),
            jax.ShapeDtypeStruct((_E // 8, 8), jnp.float32),
        ],
    )(feats8, m, b.reshape(1, 1))
    rewards = rew8.reshape(_E)
    exp_rewards = exw8.reshape(_E)

    mesh = plsc.VectorSubcoreMesh(
        core_axis_name="c", subcore_axis_name="s", num_cores=1)
    fp = pl.kernel(
        _fp_body,
        out_type=[
            jax.ShapeDtypeStruct((_NP,), jnp.float32),
            jax.ShapeDtypeStruct((_E,), jnp.float32),
        ],
        mesh=mesh,
        scratch_types=[
            pltpu.VMEM((_C,), jnp.int32),
            pltpu.VMEM((_C,), jnp.int32),
            pltpu.VMEM((_C,), jnp.float32),
            pltpu.VMEM((_C,), jnp.float32),
            pltpu.VMEM((_C,), jnp.float32),
            pltpu.VMEM((_C,), jnp.int32),
            pltpu.VMEM((_C,), jnp.int32),
            pltpu.VMEM((_C,), jnp.float32),
            pltpu.VMEM((_C,), jnp.float32),
            pltpu.VMEM((_C,), jnp.float32),
            pltpu.VMEM((_NT,), jnp.float32),
            pltpu.VMEM((_NT,), jnp.float32),
            pltpu.SemaphoreType.DMA,
            pltpu.SemaphoreType.DMA,
            pltpu.SemaphoreType.DMA,
            pltpu.SemaphoreType.DMA,
            pltpu.SemaphoreType.DMA,
            pltpu.VMEM_SHARED((_NP,), jnp.float32),
            pltpu.VMEM_SHARED((_NP,), jnp.float32),
        ],
    )
    z_pad, edge_probs = fp(src, dst, exp_rewards, sinkf)

    values = pl.pallas_call(
        _log_body,
        out_shape=jax.ShapeDtypeStruct((784, 128), jnp.float32),
    )(z_pad.reshape(784, 128)).reshape(_NP)[:_N]

    return rewards, values, edge_probs


# speedup vs baseline: 191.7112x; 1.1241x over previous
import functools

import jax
import jax.numpy as jnp
from jax import lax
from jax.experimental import pallas as pl
from jax.experimental.pallas import tpu as pltpu
from jax.experimental.pallas import tpu_sc as plsc

_N = 100000
_E = 3200000
_D = 16
_ITERS = 60
_NS = 16
_NP = 100352
_NT = _NP // _NS
_ET = _E // _NS
_C = 2000
_NCHUNK = _ET // _C
_VPC = _C // 16


def _encoder_body(feats8_ref, m_ref, b_ref, rew_ref, exw_ref):
    x = jnp.dot(feats8_ref[...], m_ref[...],
                preferred_element_type=jnp.float32,
                precision=lax.Precision.HIGHEST)
    x = x + b_ref[0, 0]
    r = -jax.nn.softplus(x)
    rew_ref[...] = r
    exw_ref[...] = jnp.exp(r)


def _log_body(z_ref, out_ref):
    out_ref[...] = jnp.log(z_ref[...])


def _fp_body(src_hbm, dst_hbm, w_hbm, sinkf_hbm,
             z_hbm, probs_hbm,
             srcA, dstA, wA, tvA,
             srcB, dstB, wB, tvB,
             zrep,
             semz, semlA, semlB, semsA, semsB,
             zacc):
    tid = lax.axis_index("s")
    nbase = tid * _NT
    ebase = tid * _ET
    fq = _NT // 4

    def _lin_issue(off, sv, dv, wvv, sem):
        pltpu.async_copy(src_hbm.at[pl.ds(off, _C)], sv, sem)
        pltpu.async_copy(dst_hbm.at[pl.ds(off, _C)], dv, sem)
        pltpu.async_copy(w_hbm.at[pl.ds(off, _C)], wvv, sem)

    def _lin_wait(off, sv, dv, wvv, sem):
        pltpu.make_async_copy(src_hbm.at[pl.ds(off, _C)], sv, sem).wait()
        pltpu.make_async_copy(dst_hbm.at[pl.ds(off, _C)], dv, sem).wait()
        pltpu.make_async_copy(w_hbm.at[pl.ds(off, _C)], wvv, sem).wait()

    def _gather_mul(tvv, wvv, dvv):
        def _m(i, c2):
            s = pl.ds(i * 16, 16)
            zd = plsc.load_gather(zrep, [dvv[s]])
            tvv[s] = wvv[s] * zd
            return c2

        lax.fori_loop(0, _VPC, _m, 0)

    def _zero_tvA(i, c):
        tvA[pl.ds(i * 16, 16)] = jnp.zeros((16,), jnp.float32)
        return c

    def _initq(q, c):
        sl = pl.ds(nbase + q * fq, fq)
        sub = pl.ds(0, fq)
        pltpu.sync_copy(sinkf_hbm.at[sl], tvA.at[sub])
        pltpu.sync_copy(tvA.at[sub], z_hbm.at[sl])
        lax.fori_loop(0, fq // 16, _zero_tvA, 0)
        pltpu.sync_copy(tvA.at[sub], zacc.at[sl])
        return c

    lax.fori_loop(0, 4, _initq, 0)
    plsc.subcore_barrier()
    pltpu.sync_copy(z_hbm.at[pl.ds(0, _N)], zrep)

    def _iter(it, c):
        _lin_issue(ebase, srcA, dstA, wA, semlA)

        def _chunk2(j, cc):
            kA = 2 * j
            kB = 2 * j + 1
            offA = ebase + kA * _C
            offB = ebase + kB * _C
            _lin_wait(offA, srcA, dstA, wA, semlA)

            @pl.when(j > 0)
            def _():
                pltpu.make_async_copy(tvB, zacc.at[srcB], semsB).wait()

            _lin_issue(offB, srcB, dstB, wB, semlB)
            _gather_mul(tvA, wA, dstA)
            dA = pltpu.async_copy(tvA, zacc.at[srcA], semsA, add=True)
            _lin_wait(offB, srcB, dstB, wB, semlB)
            dA.wait()

            @pl.when(kB + 1 < _NCHUNK)
            def _():
                _lin_issue(ebase + (kB + 1) * _C, srcA, dstA, wA, semlA)

            _gather_mul(tvB, wB, dstB)
            pltpu.async_copy(tvB, zacc.at[srcB], semsB, add=True)
            return cc

        lax.fori_loop(0, _NCHUNK // 2, _chunk2, 0)
        pltpu.make_async_copy(tvB, zacc.at[srcB], semsB).wait()
        plsc.subcore_barrier()

        def _finq(q, c2):
            sl = pl.ds(nbase + q * fq, fq)
            sub = pl.ds(0, fq)
            pltpu.sync_copy(zacc.at[sl], tvA.at[sub])
            pltpu.sync_copy(sinkf_hbm.at[sl], wA.at[sub])

            def _fin(i, c3):
                s = pl.ds(i * 16, 16)
                sf = wA[s]
                tvA[s] = sf + (1.0 - sf) * tvA[s]
                return c3

            lax.fori_loop(0, fq // 16, _fin, 0)
            pltpu.sync_copy(tvA.at[sub], z_hbm.at[sl])
            lax.fori_loop(0, fq // 16, _zero_tvA, 0)
            pltpu.sync_copy(tvA.at[sub], zacc.at[sl])
            return c2

        lax.fori_loop(0, 4, _finq, 0)
        plsc.subcore_barrier()
        pltpu.sync_copy(z_hbm.at[pl.ds(0, _N)], zrep)
        return c

    lax.fori_loop(0, _ITERS, _iter, 0)

    def _pchunk(k, c):
        off = ebase + k * _C
        _lin_issue(off, srcA, dstA, wA, semlA)
        _lin_wait(off, srcA, dstA, wA, semlA)

        def _p(i, c2):
            s = pl.ds(i * 16, 16)
            zd = plsc.load_gather(zrep, [dstA[s]])
            zs = plsc.load_gather(zrep, [srcA[s]])
            tvA[s] = wA[s] * zd / zs
            return c2

        lax.fori_loop(0, _VPC, _p, 0)
        pltpu.sync_copy(tvA, probs_hbm.at[pl.ds(off, _C)])
        return c

    lax.fori_loop(0, _NCHUNK, _pchunk, 0)


def kernel(edge_index, edge_feats, sink_node_mask, W, b):
    src = edge_index[0]
    dst = edge_index[1]
    sinkf = jnp.pad(sink_node_mask.astype(jnp.float32), (0, _NP - _N))

    feats8 = edge_feats.reshape(_E // 8, 128)
    m = jnp.kron(jnp.eye(8, dtype=jnp.float32), W)
    eb8 = 3200
    rew8, exw8 = pl.pallas_call(
        _encoder_body,
        grid=(_E // 8 // eb8,),
        in_specs=[
            pl.BlockSpec((eb8, 128), lambda i: (i, 0)),
            pl.BlockSpec((128, 8), lambda i: (0, 0)),
            pl.BlockSpec((1, 1), lambda i: (0, 0)),
        ],
        out_specs=[
            pl.BlockSpec((eb8, 8), lambda i: (i, 0)),
            pl.BlockSpec((eb8, 8), lambda i: (i, 0)),
        ],
        out_shape=[
            jax.ShapeDtypeStruct((_E // 8, 8), jnp.float32),
            jax.ShapeDtypeStruct((_E // 8, 8), jnp.float32),
        ],
    )(feats8, m, b.reshape(1, 1))
    rewards = rew8.reshape(_E)
    exp_rewards = exw8.reshape(_E)

    mesh = plsc.VectorSubcoreMesh(
        core_axis_name="c", subcore_axis_name="s", num_cores=1)
    fp = pl.kernel(
        _fp_body,
        out_type=[
            jax.ShapeDtypeStruct((_NP,), jnp.float32),
            jax.ShapeDtypeStruct((_E,), jnp.float32),
        ],
        mesh=mesh,
        compiler_params=pltpu.CompilerParams(needs_layout_passes=False),
        scratch_types=[
            pltpu.VMEM((_C,), jnp.int32),
            pltpu.VMEM((_C,), jnp.int32),
            pltpu.VMEM((_C,), jnp.float32),
            pltpu.VMEM((_C,), jnp.float32),
            pltpu.VMEM((_C,), jnp.int32),
            pltpu.VMEM((_C,), jnp.int32),
            pltpu.VMEM((_C,), jnp.float32),
            pltpu.VMEM((_C,), jnp.float32),
            pltpu.VMEM((_N,), jnp.float32),
            pltpu.SemaphoreType.DMA,
            pltpu.SemaphoreType.DMA,
            pltpu.SemaphoreType.DMA,
            pltpu.SemaphoreType.DMA,
            pltpu.SemaphoreType.DMA,
            pltpu.VMEM_SHARED((_NP,), jnp.float32),
        ],
    )
    z_pad, edge_probs = fp(src, dst, exp_rewards, sinkf)

    values = pl.pallas_call(
        _log_body,
        out_shape=jax.ShapeDtypeStruct((784, 128), jnp.float32),
    )(z_pad.reshape(784, 128)).reshape(_NP)[:_N]

    return rewards, values, edge_probs


# speedup vs baseline: 225.6715x; 1.1771x over previous
import functools

import jax
import jax.numpy as jnp
from jax import lax
from jax.experimental import pallas as pl
from jax.experimental.pallas import tpu as pltpu
from jax.experimental.pallas import tpu_sc as plsc

_N = 100000
_E = 3200000
_D = 16
_ITERS = 60
_NC = 2
_NS = 16
_NP = 100352
_NT = _NP // _NS
_EH = _E // _NC
_ET = _EH // _NS
_C = 2000
_NCH_E = _ET // _C
_NCH_Z = _N // _C
_VPC = _C // 16
_FQ = _NT // 4


def _encoder_body(feats8_ref, m_ref, b_ref, rew_ref, exw_ref):
    x = jnp.dot(feats8_ref[...], m_ref[...],
                preferred_element_type=jnp.float32,
                precision=lax.Precision.HIGHEST)
    x = x + b_ref[0, 0]
    r = -jax.nn.softplus(x)
    rew_ref[...] = r
    exw_ref[...] = jnp.exp(r)


def _log_body(p_ref, out_ref):
    out_ref[...] = jnp.log(p_ref[0] + p_ref[1])


def _combine_into_zrep(pin_hbm, zrep, bufLA, bufHA, bufLB, bufHB,
                       semA, semB):
    def _issue(off, bl, bh, sem):
        pltpu.async_copy(pin_hbm.at[pl.ds(off, _C)], bl, sem)
        pltpu.async_copy(pin_hbm.at[pl.ds(_NP + off, _C)], bh, sem)

    def _wait(off, bl, bh, sem):
        pltpu.make_async_copy(pin_hbm.at[pl.ds(off, _C)], bl, sem).wait()
        pltpu.make_async_copy(pin_hbm.at[pl.ds(_NP + off, _C)], bh, sem).wait()

    def _compute(zoff, bl, bh):
        def _m(i, c):
            s = pl.ds(i * 16, 16)
            zrep[pl.ds(zoff + i * 16, 16)] = bl[s] + bh[s]
            return c

        lax.fori_loop(0, _VPC, _m, 0)

    _issue(0, bufLA, bufHA, semA)

    def _j(j, c):
        kA = 2 * j
        kB = 2 * j + 1
        _wait(kA * _C, bufLA, bufHA, semA)
        _issue(kB * _C, bufLB, bufHB, semB)
        _compute(kA * _C, bufLA, bufHA)
        _wait(kB * _C, bufLB, bufHB, semB)

        @pl.when(kB + 1 < _NCH_Z)
        def _():
            _issue((kB + 1) * _C, bufLA, bufHA, semA)

        _compute(kB * _C, bufLB, bufHB)
        return c

    lax.fori_loop(0, _NCH_Z // 2, _j, 0)


def _fp_iter_body(src_hbm, dst_hbm, w_hbm, sinkf_hbm, pin_hbm,
                  pout_hbm,
                  srcA, dstA, wA, tvA,
                  srcB, dstB, wB, tvB,
                  zrep,
                  semz, semlA, semlB, semsA, semsB,
                  zacc):
    cid = lax.axis_index("c")
    tid = lax.axis_index("s")
    ebase = cid * _EH + tid * _ET
    nsl = tid * _NT

    def _lin_issue(off, sv, dv, wvv, sem):
        pltpu.async_copy(src_hbm.at[pl.ds(off, _C)], sv, sem)
        pltpu.async_copy(dst_hbm.at[pl.ds(off, _C)], dv, sem)
        pltpu.async_copy(w_hbm.at[pl.ds(off, _C)], wvv, sem)

    def _lin_wait(off, sv, dv, wvv, sem):
        pltpu.make_async_copy(src_hbm.at[pl.ds(off, _C)], sv, sem).wait()
        pltpu.make_async_copy(dst_hbm.at[pl.ds(off, _C)], dv, sem).wait()
        pltpu.make_async_copy(w_hbm.at[pl.ds(off, _C)], wvv, sem).wait()

    def _gather_mul(tvv, wvv, dvv):
        def _m(i, c2):
            s = pl.ds(i * 16, 16)
            zd = plsc.load_gather(zrep, [dvv[s]])
            tvv[s] = wvv[s] * zd
            return c2

        lax.fori_loop(0, _VPC, _m, 0)

    _combine_into_zrep(pin_hbm, zrep, wA, tvA, wB, tvB, semlA, semlB)

    def _zero_tvA(i, c):
        tvA[pl.ds(i * 16, 16)] = jnp.zeros((16,), jnp.float32)
        return c

    lax.fori_loop(0, _FQ // 16, _zero_tvA, 0)

    def _zq(q, c):
        pltpu.sync_copy(tvA.at[pl.ds(0, _FQ)],
                        zacc.at[pl.ds(nsl + q * _FQ, _FQ)])
        return c

    lax.fori_loop(0, 4, _zq, 0)
    plsc.subcore_barrier()

    _lin_issue(ebase, srcA, dstA, wA, semlA)

    def _chunk2(j, cc):
        kA = 2 * j
        kB = 2 * j + 1
        offA = ebase + kA * _C
        offB = ebase + kB * _C
        _lin_wait(offA, srcA, dstA, wA, semlA)

        @pl.when(j > 0)
        def _():
            pltpu.make_async_copy(tvB, zacc.at[srcB], semsB).wait()

        _lin_issue(offB, srcB, dstB, wB, semlB)
        _gather_mul(tvA, wA, dstA)
        dA = pltpu.async_copy(tvA, zacc.at[srcA], semsA, add=True)
        _lin_wait(offB, srcB, dstB, wB, semlB)
        dA.wait()

        @pl.when(kB + 1 < _NCH_E)
        def _():
            _lin_issue(ebase + (kB + 1) * _C, srcA, dstA, wA, semlA)

        _gather_mul(tvB, wB, dstB)
        pltpu.async_copy(tvB, zacc.at[srcB], semsB, add=True)
        return cc

    lax.fori_loop(0, _NCH_E // 2, _chunk2, 0)
    pltpu.make_async_copy(tvB, zacc.at[srcB], semsB).wait()
    plsc.subcore_barrier()

    def _finq(q, c):
        accsl = pl.ds(nsl + q * _FQ, _FQ)
        outsl = pl.ds(cid * _NP + nsl + q * _FQ, _FQ)
        sub = pl.ds(0, _FQ)
        pltpu.sync_copy(zacc.at[accsl], tvA.at[sub])
        pltpu.sync_copy(sinkf_hbm.at[pl.ds(nsl + q * _FQ, _FQ)],
                        wB.at[sub])

        def _fin(i, c2):
            s = pl.ds(i * 16, 16)
            sf = wB[s]
            tvA[s] = sf * 0.5 + (1.0 - sf) * tvA[s]
            return c2

        lax.fori_loop(0, _FQ // 16, _fin, 0)
        pltpu.sync_copy(tvA.at[sub], pout_hbm.at[outsl])
        return c

    lax.fori_loop(0, 4, _finq, 0)


def _probs_body(src_hbm, dst_hbm, w_hbm, pin_hbm,
                probs_hbm,
                srcA, dstA, wA, tvA,
                srcB, dstB, wB, tvB,
                zrep,
                semz, semlA, semlB, semsA, semsB):
    cid = lax.axis_index("c")
    tid = lax.axis_index("s")
    ebase = cid * _EH + tid * _ET

    _combine_into_zrep(pin_hbm, zrep, wA, tvA, wB, tvB, semlA, semlB)

    def _lin_issue(off, sv, dv, wvv, sem):
        pltpu.async_copy(src_hbm.at[pl.ds(off, _C)], sv, sem)
        pltpu.async_copy(dst_hbm.at[pl.ds(off, _C)], dv, sem)
        pltpu.async_copy(w_hbm.at[pl.ds(off, _C)], wvv, sem)

    def _lin_wait(off, sv, dv, wvv, sem):
        pltpu.make_async_copy(src_hbm.at[pl.ds(off, _C)], sv, sem).wait()
        pltpu.make_async_copy(dst_hbm.at[pl.ds(off, _C)], dv, sem).wait()
        pltpu.make_async_copy(w_hbm.at[pl.ds(off, _C)], wvv, sem).wait()

    def _probs_compute(tvv, wvv, svv, dvv):
        def _p(i, c2):
            s = pl.ds(i * 16, 16)
            zd = plsc.load_gather(zrep, [dvv[s]])
            zs = plsc.load_gather(zrep, [svv[s]])
            tvv[s] = wvv[s] * zd / zs
            return c2

        lax.fori_loop(0, _VPC, _p, 0)

    _lin_issue(ebase, srcA, dstA, wA, semlA)

    def _chunk2(j, cc):
        kA = 2 * j
        kB = 2 * j + 1
        offA = ebase + kA * _C
        offB = ebase + kB * _C
        _lin_wait(offA, srcA, dstA, wA, semlA)
        _lin_issue(offB, srcB, dstB, wB, semlB)
        _probs_compute(tvA, wA, srcA, dstA)
        pltpu.sync_copy(tvA, probs_hbm.at[pl.ds(offA, _C)])
        _lin_wait(offB, srcB, dstB, wB, semlB)

        @pl.when(kB + 1 < _NCH_E)
        def _():
            _lin_issue(ebase + (kB + 1) * _C, srcA, dstA, wA, semlA)

        _probs_compute(tvB, wB, srcB, dstB)
        pltpu.sync_copy(tvB, probs_hbm.at[pl.ds(offB, _C)])
        return cc

    lax.fori_loop(0, _NCH_E // 2, _chunk2, 0)


_SC_SCRATCH = [
    pltpu.VMEM((_C,), jnp.int32),
    pltpu.VMEM((_C,), jnp.int32),
    pltpu.VMEM((_C,), jnp.float32),
    pltpu.VMEM((_C,), jnp.float32),
    pltpu.VMEM((_C,), jnp.int32),
    pltpu.VMEM((_C,), jnp.int32),
    pltpu.VMEM((_C,), jnp.float32),
    pltpu.VMEM((_C,), jnp.float32),
    pltpu.VMEM((_N,), jnp.float32),
    pltpu.SemaphoreType.DMA,
    pltpu.SemaphoreType.DMA,
    pltpu.SemaphoreType.DMA,
    pltpu.SemaphoreType.DMA,
    pltpu.SemaphoreType.DMA,
]


def kernel(edge_index, edge_feats, sink_node_mask, W, b):
    src = edge_index[0]
    dst = edge_index[1]
    sinkf = jnp.pad(sink_node_mask.astype(jnp.float32), (0, _NP - _N))

    feats8 = edge_feats.reshape(_E // 8, 128)
    m = jnp.kron(jnp.eye(8, dtype=jnp.float32), W)
    eb8 = 3200
    rew8, exw8 = pl.pallas_call(
        _encoder_body,
        grid=(_E // 8 // eb8,),
        in_specs=[
            pl.BlockSpec((eb8, 128), lambda i: (i, 0)),
            pl.BlockSpec((128, 8), lambda i: (0, 0)),
            pl.BlockSpec((1, 1), lambda i: (0, 0)),
        ],
        out_specs=[
            pl.BlockSpec((eb8, 8), lambda i: (i, 0)),
            pl.BlockSpec((eb8, 8), lambda i: (i, 0)),
        ],
        out_shape=[
            jax.ShapeDtypeStruct((_E // 8, 8), jnp.float32),
            jax.ShapeDtypeStruct((_E // 8, 8), jnp.float32),
        ],
    )(feats8, m, b.reshape(1, 1))
    rewards = rew8.reshape(_E)
    exp_rewards = exw8.reshape(_E)

    mesh = plsc.VectorSubcoreMesh(
        core_axis_name="c", subcore_axis_name="s", num_cores=_NC)
    fp_iter = pl.kernel(
        _fp_iter_body,
        out_type=jax.ShapeDtypeStruct((_NC * _NP,), jnp.float32),
        mesh=mesh,
        compiler_params=pltpu.CompilerParams(needs_layout_passes=False),
        scratch_types=_SC_SCRATCH + [pltpu.VMEM_SHARED((_NP,), jnp.float32)],
    )
    fp_probs = pl.kernel(
        _probs_body,
        out_type=jax.ShapeDtypeStruct((_E,), jnp.float32),
        mesh=mesh,
        compiler_params=pltpu.CompilerParams(needs_layout_passes=False),
        scratch_types=_SC_SCRATCH,
    )

    def _step(p, _):
        return fp_iter(src, dst, exp_rewards, sinkf, p), None

    p0 = jnp.zeros((_NC * _NP,), jnp.float32)
    p, _ = lax.scan(_step, p0, None, length=_ITERS)

    edge_probs = fp_probs(src, dst, exp_rewards, p)

    values = pl.pallas_call(
        _log_body,
        out_shape=jax.ShapeDtypeStruct((784, 128), jnp.float32),
    )(p.reshape(_NC, 784, 128)).reshape(_NP)[:_N]

    return rewards, values, edge_probs
